# baseline (speedup 1.0000x reference)
import jax
import jax.numpy as jnp
from jax import lax
from jax.experimental import pallas as pl
from jax.experimental.pallas import tpu as pltpu
from jax.experimental.pallas import tpu_sc as plsc

MAXLEN = 200
EMBED = 32

_info = plsc.get_sparse_core_info()
NC, NS = _info.num_cores, _info.num_subcores
NW = NC * NS

CHUNK = 800
GATHER = 128
NGATHER = (CHUNK + GATHER - 1) // GATHER


def _body(x_hbm, tok_hbm, pos_hbm, out_hbm,
          idx_all, buf0, buf1, pos_rep,
          idx_sem, init_sem, gat_sem, out_sem):
    n_rows = x_hbm.shape[0]
    seqlen = pos_hbm.shape[0]
    per_w = n_rows // NW
    n_chunks = per_w // CHUNK

    sid = lax.axis_index("s")
    wid = sid * NC + lax.axis_index("c")
    base = pl.multiple_of(wid * per_w, 8)

    @pl.when(sid == 0)
    def _stage():
        for k in range(CHUNK // seqlen):
            pltpu.sync_copy(pos_hbm, pos_rep.at[pl.ds(k * seqlen, seqlen)])

    plsc.subcore_barrier()

    bufs = (buf0, buf1)

    def issue_init(buf):
        pltpu.async_copy(pos_rep, buf, init_sem)

    def wait_init(buf):
        pltpu.make_async_copy(tok_hbm.at[pl.ds(0, CHUNK)], buf, init_sem).wait()

    def wait_out(buf):
        pltpu.make_async_copy(tok_hbm.at[pl.ds(0, CHUNK)], buf, out_sem).wait()

    def do_chunk(c, s, prefetch_next, first):
        buf, obuf = bufs[s], bufs[1 - s]
        off = pl.multiple_of(c * CHUNK, 8)
        wait_init(buf)
        descs = []
        for j in range(NGATHER):
            o = j * GATHER
            ln = min(GATHER, CHUNK - o)
            descs.append(pltpu.async_copy(
                tok_hbm.at[idx_all.at[pl.ds(off + o, ln)]],
                buf.at[pl.ds(o, ln)],
                gat_sem, add=True))
        if prefetch_next:
            if not first:
                wait_out(obuf)
            issue_init(obuf)
        for d in descs:
            d.wait()
        pltpu.async_copy(buf, out_hbm.at[pl.ds(base + off, CHUNK)], out_sem)

    idx_desc = pltpu.async_copy(x_hbm.at[pl.ds(base, per_w)], idx_all, idx_sem)
    issue_init(buf0)
    idx_desc.wait()

    do_chunk(0, 0, True, True)

    def pair_body(g, _):
        do_chunk(2 * g + 1, 1, True, False)
        do_chunk(2 * g + 2, 0, True, False)
        return 0

    lax.fori_loop(0, (n_chunks - 2) // 2, pair_body, 0)

    do_chunk(n_chunks - 1, 1, False, False)

    wait_out(buf0)
    wait_out(buf1)


CONV_S = 1024


def _conv_body(in_ref, out_ref):
    t = jnp.transpose(in_ref[...], (1, 0))
    u = jnp.reshape(t, (CONV_S, 4, 32))
    out_ref[...] = jnp.concatenate([u[:, j, :] for j in range(4)], axis=1)


def _convert_table(token_table):
    vocab = token_table.shape[0]
    n_out = vocab // 4
    grid = (n_out + CONV_S - 1) // CONV_S
    conv = pl.pallas_call(
        _conv_body,
        grid=(grid,),
        in_specs=[pl.BlockSpec((32, 4 * CONV_S), lambda i: (0, i))],
        out_specs=pl.BlockSpec((CONV_S, 128), lambda i: (i, 0)),
        out_shape=jax.ShapeDtypeStruct((n_out, 128), jnp.float32),
    )
    return jnp.reshape(conv(token_table.T), (vocab, 32))


NSLAB = 2


def _gather_slab(x_slab_flat, tok_rm, pos_slab, n_rows):
    mesh = plsc.VectorSubcoreMesh(core_axis_name="c", subcore_axis_name="s")
    k = pl.kernel(
        _body,
        out_type=jax.ShapeDtypeStruct((n_rows, EMBED), jnp.float32),
        mesh=mesh,
        compiler_params=pltpu.CompilerParams(use_tc_tiling_on_sc=False),
        scratch_types=[
            pltpu.VMEM((n_rows // NW,), jnp.int32),
            pltpu.VMEM((CHUNK, EMBED), jnp.float32),
            pltpu.VMEM((CHUNK, EMBED), jnp.float32),
            pltpu.VMEM_SHARED((CHUNK, EMBED), jnp.float32),
            pltpu.SemaphoreType.DMA,
            pltpu.SemaphoreType.DMA,
            pltpu.SemaphoreType.DMA,
            pltpu.SemaphoreType.DMA,
        ],
    )
    return k(x_slab_flat, tok_rm, pos_slab)


def kernel(x, token_table, pos_table):
    batch, seqlen = x.shape
    tok_rm = _convert_table(token_table)

    lslab = seqlen // NSLAB
    outs = []
    for s in range(NSLAB):
        xs = x[:, s * lslab:(s + 1) * lslab].reshape(batch * lslab)
        ps = pos_table[s * lslab:(s + 1) * lslab]
        o = _gather_slab(xs.astype(jnp.int32), tok_rm, ps, batch * lslab)
        outs.append(o.reshape(batch, lslab, EMBED))
    return jnp.concatenate(outs, axis=1) if NSLAB > 1 else outs[0]


# speedup vs baseline: 1.4092x; 1.4092x over previous
import jax
import jax.numpy as jnp
from jax import lax
from jax.experimental import pallas as pl
from jax.experimental.pallas import tpu as pltpu
from jax.experimental.pallas import tpu_sc as plsc

MAXLEN = 200
EMBED = 32

_info = plsc.get_sparse_core_info()
NC, NS = _info.num_cores, _info.num_subcores
NW = NC * NS

CHUNK = 800
GATHER = 128
_SEQ_SPLITS = [(0, 128), (128, 72)]


def _body(x_hbm, tok_hbm, pos_hbm, out_hbm,
          idx_all, buf0, buf1, pos_rep,
          idx_sem, init_sem, gat_sem, out_sem):
    n_rows = x_hbm.shape[0]
    seqlen = pos_hbm.shape[0]
    per_w = n_rows // NW
    n_chunks = per_w // CHUNK
    bchunk = CHUNK // seqlen

    sid = lax.axis_index("s")
    wid = sid * NC + lax.axis_index("c")
    base = pl.multiple_of(wid * per_w, 8)

    @pl.when(sid == 0)
    def _stage():
        for k in range(CHUNK // seqlen):
            pltpu.sync_copy(pos_hbm, pos_rep.at[k])

    plsc.subcore_barrier()

    bufs = (buf0, buf1)

    def issue_init(buf):
        pltpu.async_copy(pos_rep, buf, init_sem)

    def wait_init(buf):
        pltpu.make_async_copy(out_hbm.at[pl.ds(0, bchunk)], buf, init_sem).wait()

    def wait_out(buf):
        pltpu.make_async_copy(out_hbm.at[pl.ds(0, bchunk)], buf, out_sem).wait()

    def do_chunk(c, s, prefetch_next, first):
        buf, obuf = bufs[s], bufs[1 - s]
        off = pl.multiple_of(c * CHUNK, 8)
        wait_init(buf)
        descs = []
        for b in range(bchunk):
            for o, ln in _SEQ_SPLITS:
                descs.append(pltpu.async_copy(
                    tok_hbm.at[idx_all.at[pl.ds(off + b * seqlen + o, ln)]],
                    buf.at[b, pl.ds(o, ln)],
                    gat_sem, add=True))
        if prefetch_next:
            if not first:
                wait_out(obuf)
            issue_init(obuf)
        for d in descs:
            d.wait()
        bstart = (base + off) // seqlen
        pltpu.async_copy(buf, out_hbm.at[pl.ds(bstart, bchunk)], out_sem)

    idx_desc = pltpu.async_copy(x_hbm.at[pl.ds(base, per_w)], idx_all, idx_sem)
    issue_init(buf0)
    idx_desc.wait()

    do_chunk(0, 0, True, True)

    def pair_body(g, _):
        do_chunk(2 * g + 1, 1, True, False)
        do_chunk(2 * g + 2, 0, True, False)
        return 0

    lax.fori_loop(0, (n_chunks - 2) // 2, pair_body, 0)

    do_chunk(n_chunks - 1, 1, False, False)

    wait_out(buf0)
    wait_out(buf1)


CONV_S = 1024


def _conv_body(in_ref, out_ref):
    t = jnp.transpose(in_ref[...], (1, 0))
    u = jnp.reshape(t, (CONV_S, 4, 32))
    out_ref[...] = jnp.concatenate([u[:, j, :] for j in range(4)], axis=1)


def _convert_table(token_table):
    vocab = token_table.shape[0]
    n_out = vocab // 4
    grid = (n_out + CONV_S - 1) // CONV_S
    conv = pl.pallas_call(
        _conv_body,
        grid=(grid,),
        in_specs=[pl.BlockSpec((32, 4 * CONV_S), lambda i: (0, i))],
        out_specs=pl.BlockSpec((CONV_S, 128), lambda i: (i, 0)),
        out_shape=jax.ShapeDtypeStruct((n_out, 128), jnp.float32),
    )
    return jnp.reshape(conv(token_table.T), (vocab, 32))


def _gather_slab(x_slab_flat, tok_rm, pos_slab, n_rows, batch):
    mesh = plsc.VectorSubcoreMesh(core_axis_name="c", subcore_axis_name="s")
    seqlen = pos_slab.shape[0]
    k = pl.kernel(
        _body,
        out_type=jax.ShapeDtypeStruct((batch, seqlen, EMBED), jnp.float32),
        mesh=mesh,
        compiler_params=pltpu.CompilerParams(use_tc_tiling_on_sc=False),
        scratch_types=[
            pltpu.VMEM((n_rows // NW,), jnp.int32),
            pltpu.VMEM((CHUNK // seqlen, seqlen, EMBED), jnp.float32),
            pltpu.VMEM((CHUNK // seqlen, seqlen, EMBED), jnp.float32),
            pltpu.VMEM_SHARED((CHUNK // seqlen, seqlen, EMBED), jnp.float32),
            pltpu.SemaphoreType.DMA,
            pltpu.SemaphoreType.DMA,
            pltpu.SemaphoreType.DMA,
            pltpu.SemaphoreType.DMA,
        ],
    )
    return k(x_slab_flat, tok_rm, pos_slab)


def kernel(x, token_table, pos_table):
    batch, seqlen = x.shape
    tok_rm = _convert_table(token_table)
    xf = x.reshape(batch * seqlen).astype(jnp.int32)
    return _gather_slab(xf, tok_rm, pos_table, batch * seqlen, batch)
